# baseline (device time: 361533 ns/iter reference)
import jax
import jax.numpy as jnp
from jax import lax
from jax.experimental import pallas as pl
from jax.experimental.pallas import tpu as pltpu

N_DEV = 8
SUB = 4


def kernel(x, w_mat):
    m, k_shard = x.shape
    _, n = w_mat.shape
    m_blk = m // N_DEV
    half = n // 2
    sw = half // SUB

    def body(
        x_hbm,
        w_hbm,
        out_ref,
        x_f32,
        x_blk,
        w_ref,
        send_buf,
        recv_buf,
        x_sems,
        w_sems,
        send_sems,
        recv_sems,
    ):
        d = lax.axis_index("i")
        right = lax.rem(d + 1, N_DEV)
        left = lax.rem(d + N_DEV - 1, N_DEV)
        dir_dst = (right, left)

        barrier_sem = pltpu.get_barrier_semaphore()

        def nbr_barrier():
            for nbr in (left, right):
                pl.semaphore_signal(
                    barrier_sem,
                    inc=1,
                    device_id=(nbr,),
                    device_id_type=pl.DeviceIdType.MESH,
                )
            pl.semaphore_wait(barrier_sem, 2)

        def rdma(dirn, c, parity):
            col = pl.ds(dirn * half + c * sw, sw)
            return pltpu.make_async_remote_copy(
                src_ref=send_buf.at[:, col],
                dst_ref=recv_buf.at[parity, :, col],
                send_sem=send_sems.at[dirn, c, parity],
                recv_sem=recv_sems.at[dirn, c, parity],
                device_id=(dir_dst[dirn],),
                device_id_type=pl.DeviceIdType.MESH,
            )

        n_strips = n // sw
        for k in range(n_strips):
            stage = pl.ds((k % 2) * sw, sw)
            if k == 0:
                pltpu.make_async_copy(
                    w_hbm.at[:, pl.ds(0, sw)],
                    x_f32.at[:, stage],
                    w_sems.at[0],
                ).start()
            pltpu.make_async_copy(
                w_hbm.at[:, pl.ds(k * sw, sw)],
                x_f32.at[:, stage],
                w_sems.at[k % 2],
            ).wait()
            if k + 1 < n_strips:
                pltpu.make_async_copy(
                    w_hbm.at[:, pl.ds((k + 1) * sw, sw)],
                    x_f32.at[:, pl.ds(((k + 1) % 2) * sw, sw)],
                    w_sems.at[(k + 1) % 2],
                ).start()
            w_ref[:, pl.ds(k * sw, sw)] = x_f32[:, stage].astype(jnp.bfloat16)

        for s in range(N_DEV):
            t_cw = lax.rem(d + (N_DEV - 1 - s), N_DEV)
            t_ccw = lax.rem(d + s + 1, N_DEV)

            nbr_barrier()

            for dirn, t in ((0, t_cw), (1, t_ccw)):
                fetch = pltpu.make_async_copy(
                    x_hbm.at[pl.ds(t * m_blk, m_blk), :],
                    x_f32,
                    x_sems.at[dirn],
                )
                fetch.start()
                fetch.wait()
                x_blk[dirn, :, :] = x_f32[:, :].astype(jnp.bfloat16)

            def sub_body(c, _, s=s):
                for dirn in (0, 1):
                    col = pl.ds(dirn * half + c * sw, sw)
                    if s >= 1:
                        rdma(dirn, c, (s - 1) % 2).wait_send()
                        rdma(dirn, c, (s - 1) % 2).wait_recv()
                    contrib = jnp.dot(
                        x_blk[dirn],
                        w_ref[:, col],
                        preferred_element_type=jnp.float32,
                    )
                    if s >= 1:
                        contrib = contrib + recv_buf[
                            (s - 1) % 2, :, col
                        ].astype(jnp.float32)
                    if s < N_DEV - 1:
                        send_buf[:, col] = contrib.astype(jnp.bfloat16)
                        rdma(dirn, c, s % 2).start()
                    else:
                        out_ref[:, col] = contrib
                return 0

            lax.fori_loop(0, SUB, sub_body, 0)

    return pl.pallas_call(
        body,
        out_shape=jax.ShapeDtypeStruct((m_blk, n), jnp.float32),
        in_specs=[
            pl.BlockSpec(memory_space=pl.ANY),
            pl.BlockSpec(memory_space=pl.ANY),
        ],
        out_specs=pl.BlockSpec(memory_space=pltpu.VMEM),
        scratch_shapes=[
            pltpu.VMEM((m_blk, k_shard), jnp.float32),
            pltpu.VMEM((2, m_blk, k_shard), jnp.bfloat16),
            pltpu.VMEM((k_shard, n), jnp.bfloat16),
            pltpu.VMEM((m_blk, n), jnp.bfloat16),
            pltpu.VMEM((2, m_blk, n), jnp.bfloat16),
            pltpu.SemaphoreType.DMA((2,)),
            pltpu.SemaphoreType.DMA((2,)),
            pltpu.SemaphoreType.DMA((2, SUB, 2)),
            pltpu.SemaphoreType.DMA((2, SUB, 2)),
        ],
        compiler_params=pltpu.CompilerParams(
            collective_id=0,
            vmem_limit_bytes=62 * 1024 * 1024,
        ),
    )(x, w_mat)


# device time: 350315 ns/iter; 1.0320x vs baseline; 1.0320x over previous
import jax
import jax.numpy as jnp
from jax import lax
from jax.experimental import pallas as pl
from jax.experimental.pallas import tpu as pltpu

N_DEV = 8
SUB = 4


def kernel(x, w_mat):
    m, k_shard = x.shape
    _, n = w_mat.shape
    m_blk = m // N_DEV
    half = n // 2
    sw = half // SUB

    def body(
        x_hbm,
        w_hbm,
        out_ref,
        x_f32,
        x_blk,
        w_ref,
        send_buf,
        recv_buf,
        x_sems,
        w_sems,
        send_sems,
        recv_sems,
    ):
        d = lax.axis_index("i")
        right = lax.rem(d + 1, N_DEV)
        left = lax.rem(d + N_DEV - 1, N_DEV)
        dir_dst = (right, left)

        barrier_sem = pltpu.get_barrier_semaphore()

        def nbr_barrier():
            for nbr in (left, right):
                pl.semaphore_signal(
                    barrier_sem,
                    inc=1,
                    device_id=(nbr,),
                    device_id_type=pl.DeviceIdType.MESH,
                )
            pl.semaphore_wait(barrier_sem, 2)

        def rdma(dirn, c, parity):
            col = pl.ds(dirn * half + c * sw, sw)
            return pltpu.make_async_remote_copy(
                src_ref=send_buf.at[:, col],
                dst_ref=recv_buf.at[parity, :, col],
                send_sem=send_sems.at[dirn, c, parity],
                recv_sem=recv_sems.at[dirn, c, parity],
                device_id=(dir_dst[dirn],),
                device_id_type=pl.DeviceIdType.MESH,
            )

        def w_strip_copy(cols, dirn):
            return pltpu.make_async_copy(
                w_hbm.at[:, cols],
                x_f32.at[:, pl.ds(dirn * sw, sw)],
                w_sems.at[dirn],
            )

        for s in range(N_DEV):
            t_cw = lax.rem(d + (N_DEV - 1 - s), N_DEV)
            t_ccw = lax.rem(d + s + 1, N_DEV)

            nbr_barrier()

            for dirn, t in ((0, t_cw), (1, t_ccw)):
                fetch = pltpu.make_async_copy(
                    x_hbm.at[pl.ds(t * m_blk, m_blk), :],
                    x_f32,
                    x_sems.at[dirn],
                )
                fetch.start()
                fetch.wait()
                x_blk[dirn, :, :] = x_f32[:, :].astype(jnp.bfloat16)

            if s == 0:
                for dirn in (0, 1):
                    w_strip_copy(pl.ds(dirn * half, sw), dirn).start()

            def sub_body(c, _, s=s):
                for dirn in (0, 1):
                    col = pl.ds(dirn * half + c * sw, sw)
                    if s == 0:
                        w_strip_copy(col, dirn).wait()
                        w_ref[:, col] = x_f32[
                            :, pl.ds(dirn * sw, sw)
                        ].astype(jnp.bfloat16)

                        @pl.when(c + 1 < SUB)
                        def _():
                            w_strip_copy(
                                pl.ds(dirn * half + (c + 1) * sw, sw), dirn
                            ).start()

                    if s >= 1:
                        rdma(dirn, c, (s - 1) % 2).wait_send()
                        rdma(dirn, c, (s - 1) % 2).wait_recv()
                    contrib = jnp.dot(
                        x_blk[dirn],
                        w_ref[:, col],
                        preferred_element_type=jnp.float32,
                    )
                    if s >= 1:
                        contrib = contrib + recv_buf[
                            (s - 1) % 2, :, col
                        ].astype(jnp.float32)
                    if s < N_DEV - 1:
                        send_buf[:, col] = contrib.astype(jnp.bfloat16)
                        rdma(dirn, c, s % 2).start()
                    else:
                        out_ref[:, col] = contrib
                return 0

            lax.fori_loop(0, SUB, sub_body, 0)

    return pl.pallas_call(
        body,
        out_shape=jax.ShapeDtypeStruct((m_blk, n), jnp.float32),
        in_specs=[
            pl.BlockSpec(memory_space=pl.ANY),
            pl.BlockSpec(memory_space=pl.ANY),
        ],
        out_specs=pl.BlockSpec(memory_space=pltpu.VMEM),
        scratch_shapes=[
            pltpu.VMEM((m_blk, k_shard), jnp.float32),
            pltpu.VMEM((2, m_blk, k_shard), jnp.bfloat16),
            pltpu.VMEM((k_shard, n), jnp.bfloat16),
            pltpu.VMEM((m_blk, n), jnp.bfloat16),
            pltpu.VMEM((2, m_blk, n), jnp.bfloat16),
            pltpu.SemaphoreType.DMA((2,)),
            pltpu.SemaphoreType.DMA((2,)),
            pltpu.SemaphoreType.DMA((2, SUB, 2)),
            pltpu.SemaphoreType.DMA((2, SUB, 2)),
        ],
        compiler_params=pltpu.CompilerParams(
            collective_id=0,
            vmem_limit_bytes=62 * 1024 * 1024,
        ),
    )(x, w_mat)


# device time: 349642 ns/iter; 1.0340x vs baseline; 1.0019x over previous
import jax
import jax.numpy as jnp
from jax import lax
from jax.experimental import pallas as pl
from jax.experimental.pallas import tpu as pltpu

N_DEV = 8
SUB = 8


def kernel(x, w_mat):
    m, k_shard = x.shape
    _, n = w_mat.shape
    m_blk = m // N_DEV
    half = n // 2
    sw = half // SUB

    def body(
        x_hbm,
        w_hbm,
        out_ref,
        x_f32,
        x_blk,
        w_ref,
        send_buf,
        recv_buf,
        x_sems,
        w_sems,
        send_sems,
        recv_sems,
    ):
        d = lax.axis_index("i")
        right = lax.rem(d + 1, N_DEV)
        left = lax.rem(d + N_DEV - 1, N_DEV)
        dir_dst = (right, left)

        barrier_sem = pltpu.get_barrier_semaphore()

        def nbr_barrier():
            for nbr in (left, right):
                pl.semaphore_signal(
                    barrier_sem,
                    inc=1,
                    device_id=(nbr,),
                    device_id_type=pl.DeviceIdType.MESH,
                )
            pl.semaphore_wait(barrier_sem, 2)

        def rdma(dirn, c, parity):
            col = pl.ds(dirn * half + c * sw, sw)
            return pltpu.make_async_remote_copy(
                src_ref=send_buf.at[:, col],
                dst_ref=recv_buf.at[parity, :, col],
                send_sem=send_sems.at[dirn, c, parity],
                recv_sem=recv_sems.at[dirn, c, parity],
                device_id=(dir_dst[dirn],),
                device_id_type=pl.DeviceIdType.MESH,
            )

        def w_strip_copy(cols, dirn):
            return pltpu.make_async_copy(
                w_hbm.at[:, cols],
                x_f32.at[:, pl.ds(dirn * sw, sw)],
                w_sems.at[dirn],
            )

        for s in range(N_DEV):
            t_cw = lax.rem(d + (N_DEV - 1 - s), N_DEV)
            t_ccw = lax.rem(d + s + 1, N_DEV)

            nbr_barrier()

            for dirn, t in ((0, t_cw), (1, t_ccw)):
                fetch = pltpu.make_async_copy(
                    x_hbm.at[pl.ds(t * m_blk, m_blk), :],
                    x_f32,
                    x_sems.at[dirn],
                )
                fetch.start()
                fetch.wait()
                x_blk[dirn, :, :] = x_f32[:, :].astype(jnp.bfloat16)

            if s == 0:
                for dirn in (0, 1):
                    w_strip_copy(pl.ds(dirn * half, sw), dirn).start()

            def sub_body(c, _, s=s):
                for dirn in (0, 1):
                    col = pl.ds(dirn * half + c * sw, sw)
                    if s == 0:
                        w_strip_copy(col, dirn).wait()
                        w_ref[:, col] = x_f32[
                            :, pl.ds(dirn * sw, sw)
                        ].astype(jnp.bfloat16)

                        @pl.when(c + 1 < SUB)
                        def _():
                            w_strip_copy(
                                pl.ds(dirn * half + (c + 1) * sw, sw), dirn
                            ).start()

                    if s >= 1:
                        rdma(dirn, c, (s - 1) % 2).wait_send()
                        rdma(dirn, c, (s - 1) % 2).wait_recv()
                    contrib = jnp.dot(
                        x_blk[dirn],
                        w_ref[:, col],
                        preferred_element_type=jnp.float32,
                    )
                    if s >= 1:
                        contrib = contrib + recv_buf[
                            (s - 1) % 2, :, col
                        ].astype(jnp.float32)
                    if s < N_DEV - 1:
                        send_buf[:, col] = contrib.astype(jnp.bfloat16)
                        rdma(dirn, c, s % 2).start()
                    else:
                        out_ref[:, col] = contrib
                return 0

            lax.fori_loop(0, SUB, sub_body, 0)

    return pl.pallas_call(
        body,
        out_shape=jax.ShapeDtypeStruct((m_blk, n), jnp.float32),
        in_specs=[
            pl.BlockSpec(memory_space=pl.ANY),
            pl.BlockSpec(memory_space=pl.ANY),
        ],
        out_specs=pl.BlockSpec(memory_space=pltpu.VMEM),
        scratch_shapes=[
            pltpu.VMEM((m_blk, k_shard), jnp.float32),
            pltpu.VMEM((2, m_blk, k_shard), jnp.bfloat16),
            pltpu.VMEM((k_shard, n), jnp.bfloat16),
            pltpu.VMEM((m_blk, n), jnp.bfloat16),
            pltpu.VMEM((2, m_blk, n), jnp.bfloat16),
            pltpu.SemaphoreType.DMA((2,)),
            pltpu.SemaphoreType.DMA((2,)),
            pltpu.SemaphoreType.DMA((2, SUB, 2)),
            pltpu.SemaphoreType.DMA((2, SUB, 2)),
        ],
        compiler_params=pltpu.CompilerParams(
            collective_id=0,
            vmem_limit_bytes=62 * 1024 * 1024,
        ),
    )(x, w_mat)
